# baseline (speedup 1.0000x reference)
import functools

import jax
import jax.numpy as jnp
from jax import lax
from jax.experimental import pallas as pl
from jax.experimental.pallas import tpu as pltpu
from jax.experimental.pallas import tpu_sc as plsc

N_USERS = 1000000
N_ITEMS = 1000000
E_SIZE = 32
BATCH = 16384

NC = 2
NS = 16
L = 16
NW = NC * NS
B_PER_W = BATCH // NW


def _mf_score_kernel(user_hbm, item_hbm, ue_hbm, ie_hbm, ub_hbm, ib_hbm,
                     out_hbm, uidx_v, iidx_v, urows_v, irows_v, ub_v, ib_v,
                     out_v, sem):
    wid = lax.axis_index("s") * NC + lax.axis_index("c")
    base = wid * B_PER_W

    pltpu.sync_copy(user_hbm.at[pl.ds(base, B_PER_W)], uidx_v)
    pltpu.sync_copy(item_hbm.at[pl.ds(base, B_PER_W)], iidx_v)

    c1 = pltpu.async_copy(ue_hbm.at[uidx_v], urows_v, sem)
    c2 = pltpu.async_copy(ie_hbm.at[iidx_v], irows_v, sem)
    c3 = pltpu.async_copy(ub_hbm.at[uidx_v], ub_v, sem)
    c4 = pltpu.async_copy(ib_hbm.at[iidx_v], ib_v, sem)
    c1.wait()
    c2.wait()
    c3.wait()
    c4.wait()

    def chunk_body(ci, _):
        row0 = ci * L
        rows = row0 + lax.iota(jnp.int32, L)
        acc = ub_v[pl.ds(row0, L)] + ib_v[pl.ds(row0, L)]
        for d in range(E_SIZE):
            dcol = jnp.full((L,), d, jnp.int32)
            ucol = plsc.load_gather(urows_v, [rows, dcol])
            icol = plsc.load_gather(irows_v, [rows, dcol])
            acc = acc + ucol * icol
        out_v[pl.ds(row0, L)] = acc
        return 0

    lax.fori_loop(0, B_PER_W // L, chunk_body, 0)

    pltpu.sync_copy(out_v, out_hbm.at[pl.ds(base, B_PER_W)])


@jax.jit
def kernel(user, item, u_embed, i_embed, u_bias, i_bias):
    mesh = plsc.VectorSubcoreMesh(core_axis_name="c", subcore_axis_name="s")
    k = functools.partial(
        pl.kernel,
        out_type=jax.ShapeDtypeStruct((BATCH,), jnp.float32),
        mesh=mesh,
        scratch_types=[
            pltpu.VMEM((B_PER_W,), jnp.int32),
            pltpu.VMEM((B_PER_W,), jnp.int32),
            pltpu.VMEM((B_PER_W, E_SIZE), jnp.float32),
            pltpu.VMEM((B_PER_W, E_SIZE), jnp.float32),
            pltpu.VMEM((B_PER_W,), jnp.float32),
            pltpu.VMEM((B_PER_W,), jnp.float32),
            pltpu.VMEM((B_PER_W,), jnp.float32),
            pltpu.SemaphoreType.DMA,
        ],
        compiler_params=pltpu.CompilerParams(
            needs_layout_passes=False, use_tc_tiling_on_sc=False),
    )(_mf_score_kernel)
    return k(user.astype(jnp.int32), item.astype(jnp.int32),
             u_embed, i_embed,
             u_bias.reshape(-1), i_bias.reshape(-1))
